# baseline (speedup 1.0000x reference)
import functools

import jax
import jax.numpy as jnp
from jax import lax
from jax.experimental import pallas as pl
from jax.experimental.pallas import tpu as pltpu, tpu_sc as plsc

NC = 2
NS = 16
NW = NC * NS
B = 16384
D = 32
BPW = B // NW
NCHUNK = BPW // 128
NGRP = BPW // 16


def _sc_gather_reduce(user_r, item_r, neg_r, item_bais, user_laten, item_laten):
    mesh = plsc.VectorSubcoreMesh(core_axis_name="c", subcore_axis_name="s")

    @functools.partial(
        pl.kernel,
        out_type=[
            jax.ShapeDtypeStruct((B,), jnp.float32),
            jax.ShapeDtypeStruct((B,), jnp.float32),
            jax.ShapeDtypeStruct((B,), jnp.float32),
            jax.ShapeDtypeStruct((NW, 16), jnp.float32),
        ],
        mesh=mesh,
        compiler_params=pltpu.CompilerParams(
            needs_layout_passes=False, use_tc_tiling_on_sc=False),
        scratch_types=[
            pltpu.VMEM((NCHUNK, 128), jnp.int32),
            pltpu.VMEM((NCHUNK, 128), jnp.int32),
            pltpu.VMEM((NCHUNK, 128), jnp.int32),
            pltpu.VMEM((BPW, D), jnp.float32),
            pltpu.VMEM((BPW, D), jnp.float32),
            pltpu.VMEM((BPW, D), jnp.float32),
            pltpu.VMEM((BPW,), jnp.float32),
            pltpu.VMEM((BPW,), jnp.float32),
            pltpu.VMEM((BPW,), jnp.float32),
            pltpu.VMEM((BPW,), jnp.float32),
            pltpu.VMEM((BPW,), jnp.float32),
            pltpu.VMEM((16,), jnp.float32),
            pltpu.SemaphoreType.DMA,
        ],
    )
    def k(user_h, item_h, neg_h, ibias_h, ulat_h, ilat_h,
          score_h, usq_h, isq_h, nsq_h,
          uidx, iidx, nidx, ue_v, ie_v, nie_v, ib_v, nib_v,
          score_v, usq_v, isq_v, nsq_v, sem):
        wid = lax.axis_index("s") * NC + lax.axis_index("c")
        rbase = wid * NCHUNK

        pltpu.sync_copy(user_h.at[pl.ds(rbase, NCHUNK)], uidx)
        pltpu.sync_copy(item_h.at[pl.ds(rbase, NCHUNK)], iidx)
        pltpu.sync_copy(neg_h.at[pl.ds(rbase, NCHUNK)], nidx)

        copies = []
        for j in range(NCHUNK):
            sl = pl.ds(j * 128, 128)
            copies.append(pltpu.async_copy(ulat_h.at[uidx.at[j]], ue_v.at[sl], sem))
            copies.append(pltpu.async_copy(ilat_h.at[iidx.at[j]], ie_v.at[sl], sem))
            copies.append(pltpu.async_copy(ilat_h.at[nidx.at[j]], nie_v.at[sl], sem))
            copies.append(pltpu.async_copy(ibias_h.at[iidx.at[j]], ib_v.at[sl], sem))
            copies.append(pltpu.async_copy(ibias_h.at[nidx.at[j]], nib_v.at[sl], sem))
        for c in copies:
            c.wait()

        iota16 = lax.iota(jnp.int32, 16)

        def g_body(g, nacc):
            rows = g * 16 + iota16
            s = ib_v[pl.ds(g * 16, 16)] - nib_v[pl.ds(g * 16, 16)]
            u = jnp.zeros((16,), jnp.float32)
            i2 = jnp.zeros((16,), jnp.float32)
            for d in range(D):
                cd = jnp.full((16,), d, jnp.int32)
                ue = plsc.load_gather(ue_v, [rows, cd])
                ie = plsc.load_gather(ie_v, [rows, cd])
                nie = plsc.load_gather(nie_v, [rows, cd])
                s = s + ue * (ie - nie)
                u = u + ue * ue
                i2 = i2 + ie * ie
                nacc = nacc + nie * nie
            score_v[pl.ds(g * 16, 16)] = s
            usq_v[pl.ds(g * 16, 16)] = u
            isq_v[pl.ds(g * 16, 16)] = i2
            return nacc

        nacc = lax.fori_loop(0, NGRP, g_body, jnp.zeros((16,), jnp.float32))
        nsq_v[...] = nacc

        obase = wid * BPW
        pltpu.sync_copy(score_v, score_h.at[pl.ds(obase, BPW)])
        pltpu.sync_copy(usq_v, usq_h.at[pl.ds(obase, BPW)])
        pltpu.sync_copy(isq_v, isq_h.at[pl.ds(obase, BPW)])
        pltpu.sync_copy(nsq_v, nsq_h.at[wid])

    return k(user_r, item_r, neg_r, item_bais, user_laten, item_laten)


def _tc_finish(score, usq, isq, nsq):
    def body(score_ref, usq_ref, isq_ref, nsq_ref, bpr_ref, l2_ref):
        s = score_ref[...]
        softplus = jnp.maximum(-s, 0.0) + jnp.log1p(jnp.exp(-jnp.abs(s)))
        bpr_ref[0, 0] = jnp.sum(softplus)
        l2_ref[0, 0] = (jnp.sum(jnp.sqrt(usq_ref[...]))
                        + jnp.sum(jnp.sqrt(isq_ref[...]))
                        + jnp.sqrt(jnp.sum(nsq_ref[...])))

    return pl.pallas_call(
        body,
        out_shape=[jax.ShapeDtypeStruct((1, 1), jnp.float32)] * 2,
        out_specs=[pl.BlockSpec(memory_space=pltpu.SMEM)] * 2,
    )(score, usq, isq, nsq)


def kernel(user, item, neg_item, user_bais, item_bais, user_laten, item_laten):
    user_r = user.reshape(NW * NCHUNK, 128)
    item_r = item.reshape(NW * NCHUNK, 128)
    neg_r = neg_item.reshape(NW * NCHUNK, 128)
    score, usq, isq, nsq = _sc_gather_reduce(
        user_r, item_r, neg_r, item_bais.reshape(-1), user_laten, item_laten)
    bpr, l2 = _tc_finish(score.reshape(128, 128), usq.reshape(128, 128),
                         isq.reshape(128, 128), nsq.reshape(4, 128))
    return (bpr[0, 0], l2[0, 0])


# speedup vs baseline: 1.0000x; 1.0000x over previous
import functools

import jax
import jax.numpy as jnp
from jax import lax
from jax.experimental import pallas as pl
from jax.experimental.pallas import tpu as pltpu, tpu_sc as plsc

NC = 2
NS = 16
NW = NC * NS
B = 16384
D = 32
BPW = B // NW
NPASS = 4
PR = BPW // NPASS
NGRP = PR // 16


def _sc_gather_reduce(user, item, neg, ibias, ulat4, ilat4):
    mesh = plsc.VectorSubcoreMesh(core_axis_name="c", subcore_axis_name="s")

    @functools.partial(
        pl.kernel,
        out_type=[
            jax.ShapeDtypeStruct((B,), jnp.float32),
            jax.ShapeDtypeStruct((B,), jnp.float32),
            jax.ShapeDtypeStruct((B,), jnp.float32),
            jax.ShapeDtypeStruct((NW * 16,), jnp.float32),
        ],
        mesh=mesh,
        compiler_params=pltpu.CompilerParams(needs_layout_passes=False),
        scratch_types=[
            pltpu.VMEM((BPW,), jnp.int32),
            pltpu.VMEM((BPW,), jnp.int32),
            pltpu.VMEM((BPW,), jnp.int32),
            pltpu.VMEM((BPW,), jnp.int32),
            pltpu.VMEM((BPW,), jnp.int32),
            pltpu.VMEM((BPW,), jnp.int32),
            pltpu.VMEM((2, PR, 128), jnp.float32),
            pltpu.VMEM((2, PR, 128), jnp.float32),
            pltpu.VMEM((2, PR, 128), jnp.float32),
            pltpu.VMEM((BPW,), jnp.float32),
            pltpu.VMEM((BPW,), jnp.float32),
            pltpu.VMEM((BPW,), jnp.float32),
            pltpu.VMEM((BPW,), jnp.float32),
            pltpu.VMEM((BPW,), jnp.float32),
            pltpu.VMEM((16,), jnp.float32),
            pltpu.SemaphoreType.DMA,
            pltpu.SemaphoreType.DMA,
            pltpu.SemaphoreType.DMA,
        ],
    )
    def k(user_h, item_h, neg_h, ibias_h, ulat_h, ilat_h,
          score_h, usq_h, isq_h, nsq_h,
          uflat, iflat, nflat, urow4, irow4, nrow4,
          ue_b, ie_b, nie_b, ib_v, nib_v,
          score_v, usq_v, isq_v, nsq_v, semA, semB, semb):
        wid = lax.axis_index("s") * NC + lax.axis_index("c")
        base = wid * BPW

        pltpu.sync_copy(user_h.at[pl.ds(base, BPW)], uflat)
        pltpu.sync_copy(item_h.at[pl.ds(base, BPW)], iflat)
        pltpu.sync_copy(neg_h.at[pl.ds(base, BPW)], nflat)

        bias_copies = []
        for j in range(NPASS):
            sl = pl.ds(j * PR, PR)
            bias_copies.append(
                pltpu.async_copy(ibias_h.at[iflat.at[sl]], ib_v.at[sl], semb))
            bias_copies.append(
                pltpu.async_copy(ibias_h.at[nflat.at[sl]], nib_v.at[sl], semb))

        for t in range(BPW // 16):
            sl = pl.ds(t * 16, 16)
            urow4[sl] = lax.shift_right_logical(uflat[sl], 2)
            irow4[sl] = lax.shift_right_logical(iflat[sl], 2)
            nrow4[sl] = lax.shift_right_logical(nflat[sl], 2)

        def fire(p):
            sl = pl.ds(p * PR, PR)
            sem = semA if p % 2 == 0 else semB
            buf = p % 2
            return [
                pltpu.async_copy(ulat_h.at[urow4.at[sl]], ue_b.at[buf], sem),
                pltpu.async_copy(ilat_h.at[irow4.at[sl]], ie_b.at[buf], sem),
                pltpu.async_copy(ilat_h.at[nrow4.at[sl]], nie_b.at[buf], sem),
            ]

        inflight = fire(0)
        for c in bias_copies:
            c.wait()

        iota16 = lax.iota(jnp.int32, 16)
        nacc0 = jnp.zeros((16,), jnp.float32)
        for p in range(NPASS):
            nxt = fire(p + 1) if p + 1 < NPASS else []
            for c in inflight:
                c.wait()
            inflight = nxt
            buf = p % 2
            ue_p, ie_p, nie_p = ue_b.at[buf], ie_b.at[buf], nie_b.at[buf]

            def g_body(gg, nacc, _p=p, _ue=ue_p, _ie=ie_p, _nie=nie_p):
                goff = _p * PR + gg * 16
                rows = gg * 16 + iota16
                ucol = (uflat[pl.ds(goff, 16)] & 3) * D
                icol = (iflat[pl.ds(goff, 16)] & 3) * D
                ncol = (nflat[pl.ds(goff, 16)] & 3) * D
                s = ib_v[pl.ds(goff, 16)] - nib_v[pl.ds(goff, 16)]
                u = jnp.zeros((16,), jnp.float32)
                i2 = jnp.zeros((16,), jnp.float32)
                for d in range(D):
                    ue = plsc.load_gather(_ue, [rows, ucol + d])
                    ie = plsc.load_gather(_ie, [rows, icol + d])
                    nie = plsc.load_gather(_nie, [rows, ncol + d])
                    s = s + ue * (ie - nie)
                    u = u + ue * ue
                    i2 = i2 + ie * ie
                    nacc = nacc + nie * nie
                score_v[pl.ds(goff, 16)] = s
                usq_v[pl.ds(goff, 16)] = u
                isq_v[pl.ds(goff, 16)] = i2
                return nacc

            nacc0 = lax.fori_loop(0, NGRP, g_body, nacc0)

        nsq_v[...] = nacc0
        pltpu.sync_copy(score_v, score_h.at[pl.ds(base, BPW)])
        pltpu.sync_copy(usq_v, usq_h.at[pl.ds(base, BPW)])
        pltpu.sync_copy(isq_v, isq_h.at[pl.ds(base, BPW)])
        pltpu.sync_copy(nsq_v, nsq_h.at[pl.ds(wid * 16, 16)])

    return k(user, item, neg, ibias, ulat4, ilat4)


def _tc_finish(score, usq, isq, nsq):
    def body(score_ref, usq_ref, isq_ref, nsq_ref, bpr_ref, l2_ref):
        s = score_ref[...]
        softplus = jnp.maximum(-s, 0.0) + jnp.log1p(jnp.exp(-jnp.abs(s)))
        bpr_ref[0, 0] = jnp.sum(softplus)
        l2_ref[0, 0] = (jnp.sum(jnp.sqrt(usq_ref[...]))
                        + jnp.sum(jnp.sqrt(isq_ref[...]))
                        + jnp.sqrt(jnp.sum(nsq_ref[...])))

    return pl.pallas_call(
        body,
        out_shape=[jax.ShapeDtypeStruct((1, 1), jnp.float32)] * 2,
        out_specs=[pl.BlockSpec(memory_space=pltpu.SMEM)] * 2,
    )(score, usq, isq, nsq)


def kernel(user, item, neg_item, user_bais, item_bais, user_laten, item_laten):
    ulat4 = user_laten.reshape(-1, 128)
    ilat4 = item_laten.reshape(-1, 128)
    score, usq, isq, nsq = _sc_gather_reduce(
        user, item, neg_item, item_bais.reshape(-1), ulat4, ilat4)
    bpr, l2 = _tc_finish(score.reshape(128, 128), usq.reshape(128, 128),
                         isq.reshape(128, 128), nsq.reshape(4, 128))
    return (bpr[0, 0], l2[0, 0])
